# baseline (speedup 1.0000x reference)
import functools

import jax
import jax.numpy as jnp
from jax import lax
from jax.experimental import pallas as pl
from jax.experimental.pallas import tpu as pltpu
from jax.experimental.pallas import tpu_sc as plsc

N = 10000
N_PAD = 10240
E = 320000
E_PAD = 327680
CPW = 80
FILL = N + 16
NROWS = N_PAD // 16
BR = 1024

_MESH = plsc.VectorSubcoreMesh(
    core_axis_name="c", subcore_axis_name="s", num_cores=2, num_subcores=16)


@functools.partial(
    pl.kernel,
    out_type=jax.ShapeDtypeStruct((2, N_PAD), jnp.float32),
    mesh=_MESH,
    scratch_types=[
        pltpu.VMEM_SHARED((N_PAD,), jnp.float32),
        pltpu.VMEM((CPW, 128), jnp.int32),
        pltpu.VMEM((128,), jnp.float32),
    ],
)
def _deg(dst2, zeros1, degp, deg_sh, dstbuf, ones_v):
    c = lax.axis_index("c")
    s = lax.axis_index("s")
    wid = s * 2 + c
    for j in range(8):
        ones_v[pl.ds(j * 16, 16)] = jnp.ones((16,), jnp.float32)
    pltpu.sync_copy(zeros1, deg_sh.at[pl.ds(s * NROWS, NROWS)])
    plsc.subcore_barrier()
    pltpu.sync_copy(dst2.at[pl.ds(wid * CPW, CPW)], dstbuf)

    def body(j, carry):
        pltpu.sync_copy(ones_v, deg_sh.at[dstbuf.at[j]], add=True)
        return carry

    lax.fori_loop(0, CPW, body, 0)
    plsc.subcore_barrier()
    pltpu.sync_copy(deg_sh.at[pl.ds(s * NROWS, NROWS)],
                    degp.at[c, pl.ds(s * NROWS, NROWS)])


@functools.partial(
    pl.kernel,
    out_type=jax.ShapeDtypeStruct((2, N_PAD, 128), jnp.float32),
    mesh=_MESH,
    scratch_types=[
        pltpu.VMEM_SHARED((N_PAD, 128), jnp.float32),
        pltpu.VMEM((16, 128), jnp.int32),
        pltpu.VMEM((16, 128), jnp.int32),
        pltpu.VMEM((128, 128), jnp.float32),
        pltpu.VMEM((128, 128), jnp.float32),
        pltpu.SemaphoreType.DMA,
        pltpu.SemaphoreType.DMA,
    ],
)
def _agg(src2, dst2, y, zrows, sp, acc_sh, sidx, didx, r0, r1, m0, m1):
    c = lax.axis_index("c")
    s = lax.axis_index("s")
    rows = (r0, r1)
    sems = (m0, m1)
    pltpu.sync_copy(zrows, acc_sh.at[pl.ds(s * NROWS, NROWS)])
    plsc.subcore_barrier()

    nst = jnp.where(c == 0, 8, 2)

    def stage(t, carry):
        base = s * CPW * 2 + c * 128 + t * 16
        pltpu.sync_copy(src2.at[pl.ds(base, 16)], sidx)
        pltpu.sync_copy(dst2.at[pl.ds(base, 16)], didx)

        def body(k, c2):
            descs = []
            for b in range(2):
                descs.append(
                    pltpu.async_copy(y.at[sidx.at[k * 2 + b]], rows[b],
                                     sems[b]))
            for b in range(2):
                descs[b].wait()
                pltpu.sync_copy(rows[b], acc_sh.at[didx.at[k * 2 + b]],
                                add=True)
            return c2

        lax.fori_loop(0, 8, body, carry)
        return carry

    lax.fori_loop(0, nst, stage, 0)
    plsc.subcore_barrier()
    pltpu.sync_copy(acc_sh.at[pl.ds(s * NROWS, NROWS)],
                    sp.at[c, pl.ds(s * NROWS, NROWS)])


def _k1_body(deg_ref, x_ref, dinv_ref, y_ref):
    deg = deg_ref[0] + deg_ref[1] + 1.0
    dinv = lax.rsqrt(deg)
    dinv_ref[...] = dinv
    y_ref[...] = x_ref[...] * dinv


def _k1(deg3, x_pad):
    return pl.pallas_call(
        _k1_body,
        grid=(N_PAD // BR,),
        in_specs=[
            pl.BlockSpec((2, BR, 1), lambda i: (0, i, 0)),
            pl.BlockSpec((BR, 128), lambda i: (i, 0)),
        ],
        out_specs=[
            pl.BlockSpec((BR, 1), lambda i: (i, 0)),
            pl.BlockSpec((BR, 128), lambda i: (i, 0)),
        ],
        out_shape=[
            jax.ShapeDtypeStruct((N_PAD, 1), jnp.float32),
            jax.ShapeDtypeStruct((N_PAD, 128), jnp.float32),
        ],
    )(deg3, x_pad)


def _k2_body(sp_ref, y_ref, dinv_ref, w_ref, b_ref, out_ref):
    dinv = dinv_ref[...]
    z = dinv * (sp_ref[0] + sp_ref[1] + y_ref[...])
    h = jnp.dot(z, w_ref[...], preferred_element_type=jnp.float32) + b_ref[...]
    out_ref[...] = dinv * jnp.maximum(h, 0.0)


def _k2(sp, y, dinv2, w, b2d):
    return pl.pallas_call(
        _k2_body,
        grid=(N_PAD // BR,),
        in_specs=[
            pl.BlockSpec((2, BR, 128), lambda i: (0, i, 0)),
            pl.BlockSpec((BR, 128), lambda i: (i, 0)),
            pl.BlockSpec((BR, 1), lambda i: (i, 0)),
            pl.BlockSpec((128, 128), lambda i: (0, 0)),
            pl.BlockSpec((1, 128), lambda i: (0, 0)),
        ],
        out_specs=pl.BlockSpec((BR, 128), lambda i: (i, 0)),
        out_shape=jax.ShapeDtypeStruct((N_PAD, 128), jnp.float32),
    )(sp, y, dinv2, w, b2d)


def _k3_body(sp_ref, y_ref, dinv_ref, wm_ref, bm_ref, wl_ref, bl_ref,
             mu_ref, ls_ref):
    dinv = dinv_ref[...]
    z = dinv * (sp_ref[0] + sp_ref[1] + y_ref[...])
    mu_ref[...] = (
        jnp.dot(z, wm_ref[...], preferred_element_type=jnp.float32)
        + bm_ref[...])
    ls_ref[...] = (
        jnp.dot(z, wl_ref[...], preferred_element_type=jnp.float32)
        + bl_ref[...])


def _k3(sp, y, dinv2, wm, bm2, wl, bl2):
    return pl.pallas_call(
        _k3_body,
        grid=(N_PAD // BR,),
        in_specs=[
            pl.BlockSpec((2, BR, 128), lambda i: (0, i, 0)),
            pl.BlockSpec((BR, 128), lambda i: (i, 0)),
            pl.BlockSpec((BR, 1), lambda i: (i, 0)),
            pl.BlockSpec((128, 64), lambda i: (0, 0)),
            pl.BlockSpec((1, 64), lambda i: (0, 0)),
            pl.BlockSpec((128, 64), lambda i: (0, 0)),
            pl.BlockSpec((1, 64), lambda i: (0, 0)),
        ],
        out_specs=[
            pl.BlockSpec((BR, 64), lambda i: (i, 0)),
            pl.BlockSpec((BR, 64), lambda i: (i, 0)),
        ],
        out_shape=[
            jax.ShapeDtypeStruct((N_PAD, 64), jnp.float32),
            jax.ShapeDtypeStruct((N_PAD, 64), jnp.float32),
        ],
    )(sp, y, dinv2, wm, bm2, wl, bl2)


def kernel(x, edge_index, W1, b1, W2, b2, W_mu, b_mu, W_ls, b_ls):
    src = edge_index[0]
    dst = edge_index[1]
    fill = jnp.int32(FILL)
    src2 = jnp.full((E_PAD,), fill, jnp.int32).at[:E].set(src)
    src2 = src2.reshape(E_PAD // 128, 128)
    dst2 = jnp.full((E_PAD,), fill, jnp.int32).at[:E].set(dst)
    dst2 = dst2.reshape(E_PAD // 128, 128)
    x_pad = jnp.zeros((N_PAD, 128), jnp.float32).at[:N].set(x)
    zeros1 = jnp.zeros((NROWS,), jnp.float32)
    zrows = jnp.zeros((NROWS, 128), jnp.float32)

    degp = _deg(dst2, zeros1)
    dinv2, y0 = _k1(degp.reshape(2, N_PAD, 1), x_pad)
    s0 = _agg(src2, dst2, y0, zrows)
    y1 = _k2(s0, y0, dinv2, W1, b1.reshape(1, 128))
    s1 = _agg(src2, dst2, y1, zrows)
    y2 = _k2(s1, y1, dinv2, W2, b2.reshape(1, 128))
    s2 = _agg(src2, dst2, y2, zrows)
    mu, ls = _k3(s2, y2, dinv2, W_mu, b_mu.reshape(1, 64),
                 W_ls, b_ls.reshape(1, 64))
    return mu[:N], ls[:N]


# speedup vs baseline: 1.0912x; 1.0912x over previous
import functools

import jax
import jax.numpy as jnp
from jax import lax
from jax.experimental import pallas as pl
from jax.experimental.pallas import tpu as pltpu
from jax.experimental.pallas import tpu_sc as plsc

N = 10000
N_PAD = 10240
E = 320000
E_PAD = 327680
CPW = 80
FILL = N + 16
NROWS = N_PAD // 16
BR = 1024

_MESH = plsc.VectorSubcoreMesh(
    core_axis_name="c", subcore_axis_name="s", num_cores=2, num_subcores=16)


@functools.partial(
    pl.kernel,
    out_type=jax.ShapeDtypeStruct((2, N_PAD), jnp.float32),
    mesh=_MESH,
    scratch_types=[
        pltpu.VMEM_SHARED((N_PAD,), jnp.float32),
        pltpu.VMEM((CPW, 128), jnp.int32),
        pltpu.VMEM((128,), jnp.float32),
    ],
)
def _deg(dst2, zeros1, degp, deg_sh, dstbuf, ones_v):
    c = lax.axis_index("c")
    s = lax.axis_index("s")
    wid = s * 2 + c
    for j in range(8):
        ones_v[pl.ds(j * 16, 16)] = jnp.ones((16,), jnp.float32)
    pltpu.sync_copy(zeros1, deg_sh.at[pl.ds(s * NROWS, NROWS)])
    plsc.subcore_barrier()
    pltpu.sync_copy(dst2.at[pl.ds(wid * CPW, CPW)], dstbuf)

    def body(j, carry):
        pltpu.sync_copy(ones_v, deg_sh.at[dstbuf.at[j]], add=True)
        return carry

    lax.fori_loop(0, CPW, body, 0)
    plsc.subcore_barrier()
    pltpu.sync_copy(deg_sh.at[pl.ds(s * NROWS, NROWS)],
                    degp.at[c, pl.ds(s * NROWS, NROWS)])


@functools.partial(
    pl.kernel,
    out_type=jax.ShapeDtypeStruct((2, N_PAD, 128), jnp.float32),
    mesh=_MESH,
    scratch_types=[
        pltpu.VMEM_SHARED((N_PAD, 128), jnp.float32),
        pltpu.VMEM((16, 128), jnp.int32),
        pltpu.VMEM((16, 128), jnp.int32),
        pltpu.VMEM((128, 128), jnp.float32),
        pltpu.VMEM((128, 128), jnp.float32),
        pltpu.SemaphoreType.DMA,
        pltpu.SemaphoreType.DMA,
        pltpu.SemaphoreType.DMA,
        pltpu.SemaphoreType.DMA,
    ],
)
def _agg(src2, dst2, y, zrows, sp, acc_sh, sidx, didx, r0, r1,
         g0, g1, sc0, sc1):
    c = lax.axis_index("c")
    s = lax.axis_index("s")
    rows = (r0, r1)
    gsem = (g0, g1)
    ssem = (sc0, sc1)
    pltpu.sync_copy(zrows, acc_sh.at[pl.ds(s * NROWS, NROWS)])
    plsc.subcore_barrier()

    nst = jnp.where(c == 0, 8, 2)

    def stage(t, carry):
        base = s * CPW * 2 + c * 128 + t * 16
        pltpu.sync_copy(src2.at[pl.ds(base, 16)], sidx)
        pltpu.sync_copy(dst2.at[pl.ds(base, 16)], didx)
        sdescs = [None, None]
        gdescs = [None, None]
        gdescs[0] = pltpu.async_copy(y.at[sidx.at[0]], rows[0], gsem[0])
        for j in range(16):
            b = j & 1
            nb = b ^ 1
            if j + 1 < 16:
                if sdescs[nb] is not None:
                    sdescs[nb].wait()
                gdescs[nb] = pltpu.async_copy(y.at[sidx.at[j + 1]], rows[nb],
                                              gsem[nb])
            gdescs[b].wait()
            sdescs[b] = pltpu.async_copy(rows[b], acc_sh.at[didx.at[j]],
                                         ssem[b], add=True)
        sdescs[0].wait()
        sdescs[1].wait()
        return carry

    lax.fori_loop(0, nst, stage, 0)
    plsc.subcore_barrier()
    pltpu.sync_copy(acc_sh.at[pl.ds(s * NROWS, NROWS)],
                    sp.at[c, pl.ds(s * NROWS, NROWS)])


def _k1_body(deg_ref, x_ref, dinv_ref, y_ref):
    deg = deg_ref[0] + deg_ref[1] + 1.0
    dinv = lax.rsqrt(deg)
    dinv_ref[...] = dinv
    y_ref[...] = x_ref[...] * dinv


def _k1(deg3, x_pad):
    return pl.pallas_call(
        _k1_body,
        grid=(N_PAD // BR,),
        in_specs=[
            pl.BlockSpec((2, BR, 1), lambda i: (0, i, 0)),
            pl.BlockSpec((BR, 128), lambda i: (i, 0)),
        ],
        out_specs=[
            pl.BlockSpec((BR, 1), lambda i: (i, 0)),
            pl.BlockSpec((BR, 128), lambda i: (i, 0)),
        ],
        out_shape=[
            jax.ShapeDtypeStruct((N_PAD, 1), jnp.float32),
            jax.ShapeDtypeStruct((N_PAD, 128), jnp.float32),
        ],
    )(deg3, x_pad)


def _k2_body(sp_ref, y_ref, dinv_ref, w_ref, b_ref, out_ref):
    dinv = dinv_ref[...]
    z = dinv * (sp_ref[0] + sp_ref[1] + y_ref[...])
    h = jnp.dot(z, w_ref[...], preferred_element_type=jnp.float32) + b_ref[...]
    out_ref[...] = dinv * jnp.maximum(h, 0.0)


def _k2(sp, y, dinv2, w, b2d):
    return pl.pallas_call(
        _k2_body,
        grid=(N_PAD // BR,),
        in_specs=[
            pl.BlockSpec((2, BR, 128), lambda i: (0, i, 0)),
            pl.BlockSpec((BR, 128), lambda i: (i, 0)),
            pl.BlockSpec((BR, 1), lambda i: (i, 0)),
            pl.BlockSpec((128, 128), lambda i: (0, 0)),
            pl.BlockSpec((1, 128), lambda i: (0, 0)),
        ],
        out_specs=pl.BlockSpec((BR, 128), lambda i: (i, 0)),
        out_shape=jax.ShapeDtypeStruct((N_PAD, 128), jnp.float32),
    )(sp, y, dinv2, w, b2d)


def _k3_body(sp_ref, y_ref, dinv_ref, wm_ref, bm_ref, wl_ref, bl_ref,
             mu_ref, ls_ref):
    dinv = dinv_ref[...]
    z = dinv * (sp_ref[0] + sp_ref[1] + y_ref[...])
    mu_ref[...] = (
        jnp.dot(z, wm_ref[...], preferred_element_type=jnp.float32)
        + bm_ref[...])
    ls_ref[...] = (
        jnp.dot(z, wl_ref[...], preferred_element_type=jnp.float32)
        + bl_ref[...])


def _k3(sp, y, dinv2, wm, bm2, wl, bl2):
    return pl.pallas_call(
        _k3_body,
        grid=(N_PAD // BR,),
        in_specs=[
            pl.BlockSpec((2, BR, 128), lambda i: (0, i, 0)),
            pl.BlockSpec((BR, 128), lambda i: (i, 0)),
            pl.BlockSpec((BR, 1), lambda i: (i, 0)),
            pl.BlockSpec((128, 64), lambda i: (0, 0)),
            pl.BlockSpec((1, 64), lambda i: (0, 0)),
            pl.BlockSpec((128, 64), lambda i: (0, 0)),
            pl.BlockSpec((1, 64), lambda i: (0, 0)),
        ],
        out_specs=[
            pl.BlockSpec((BR, 64), lambda i: (i, 0)),
            pl.BlockSpec((BR, 64), lambda i: (i, 0)),
        ],
        out_shape=[
            jax.ShapeDtypeStruct((N_PAD, 64), jnp.float32),
            jax.ShapeDtypeStruct((N_PAD, 64), jnp.float32),
        ],
    )(sp, y, dinv2, wm, bm2, wl, bl2)


def kernel(x, edge_index, W1, b1, W2, b2, W_mu, b_mu, W_ls, b_ls):
    src = edge_index[0]
    dst = edge_index[1]
    fill = jnp.int32(FILL)
    src2 = jnp.full((E_PAD,), fill, jnp.int32).at[:E].set(src)
    src2 = src2.reshape(E_PAD // 128, 128)
    dst2 = jnp.full((E_PAD,), fill, jnp.int32).at[:E].set(dst)
    dst2 = dst2.reshape(E_PAD // 128, 128)
    x_pad = jnp.zeros((N_PAD, 128), jnp.float32).at[:N].set(x)
    zeros1 = jnp.zeros((NROWS,), jnp.float32)
    zrows = jnp.zeros((NROWS, 128), jnp.float32)

    degp = _deg(dst2, zeros1)
    dinv2, y0 = _k1(degp.reshape(2, N_PAD, 1), x_pad)
    s0 = _agg(src2, dst2, y0, zrows)
    y1 = _k2(s0, y0, dinv2, W1, b1.reshape(1, 128))
    s1 = _agg(src2, dst2, y1, zrows)
    y2 = _k2(s1, y1, dinv2, W2, b2.reshape(1, 128))
    s2 = _agg(src2, dst2, y2, zrows)
    mu, ls = _k3(s2, y2, dinv2, W_mu, b_mu.reshape(1, 64),
                 W_ls, b_ls.reshape(1, 64))
    return mu[:N], ls[:N]


# speedup vs baseline: 1.1026x; 1.0105x over previous
import functools

import jax
import jax.numpy as jnp
from jax import lax
from jax.experimental import pallas as pl
from jax.experimental.pallas import tpu as pltpu
from jax.experimental.pallas import tpu_sc as plsc

N = 10000
N_PAD = 10240
E = 320000
E_PAD = 327680
CPW = 80
FILL = N + 16
NROWS = N_PAD // 16
BR = 1024

_MESH = plsc.VectorSubcoreMesh(
    core_axis_name="c", subcore_axis_name="s", num_cores=2, num_subcores=16)


@functools.partial(
    pl.kernel,
    out_type=jax.ShapeDtypeStruct((2, N_PAD), jnp.float32),
    mesh=_MESH,
    scratch_types=[
        pltpu.VMEM_SHARED((N_PAD,), jnp.float32),
        pltpu.VMEM((160, 64), jnp.int32),
        pltpu.VMEM((64,), jnp.float32),
    ],
)
def _deg(dst2, zeros1, degp, deg_sh, dstbuf, ones_v):
    c = lax.axis_index("c")
    s = lax.axis_index("s")
    wid = s * 2 + c
    for j in range(4):
        ones_v[pl.ds(j * 16, 16)] = jnp.ones((16,), jnp.float32)
    pltpu.sync_copy(zeros1, deg_sh.at[pl.ds(s * NROWS, NROWS)])
    plsc.subcore_barrier()
    pltpu.sync_copy(dst2.at[pl.ds(wid * 160, 160)], dstbuf)

    def body(j, carry):
        pltpu.sync_copy(ones_v, deg_sh.at[dstbuf.at[j]], add=True)
        return carry

    lax.fori_loop(0, 160, body, 0)
    plsc.subcore_barrier()
    pltpu.sync_copy(deg_sh.at[pl.ds(s * NROWS, NROWS)],
                    degp.at[c, pl.ds(s * NROWS, NROWS)])


@functools.partial(
    pl.kernel,
    out_type=jax.ShapeDtypeStruct((2, N_PAD, 128), jnp.float32),
    mesh=_MESH,
    scratch_types=[
        pltpu.VMEM_SHARED((N_PAD, 128), jnp.float32),
        pltpu.VMEM((32, 64), jnp.int32),
        pltpu.VMEM((32, 64), jnp.int32),
        pltpu.VMEM((64, 128), jnp.float32),
        pltpu.VMEM((64, 128), jnp.float32),
        pltpu.VMEM((64, 128), jnp.float32),
        pltpu.VMEM((64, 128), jnp.float32),
        pltpu.SemaphoreType.DMA,
        pltpu.SemaphoreType.DMA,
        pltpu.SemaphoreType.DMA,
        pltpu.SemaphoreType.DMA,
        pltpu.SemaphoreType.DMA,
        pltpu.SemaphoreType.DMA,
        pltpu.SemaphoreType.DMA,
        pltpu.SemaphoreType.DMA,
    ],
)
def _agg(src2, dst2, y, zrows, sp, acc_sh, sidx, didx, r0, r1, r2, r3,
         g0, g1, g2, g3, sc0, sc1, sc2, sc3):
    c = lax.axis_index("c")
    s = lax.axis_index("s")
    rows = (r0, r1, r2, r3)
    gsem = (g0, g1, g2, g3)
    ssem = (sc0, sc1, sc2, sc3)
    pltpu.sync_copy(zrows, acc_sh.at[pl.ds(s * NROWS, NROWS)])
    plsc.subcore_barrier()

    nst = jnp.where(c == 0, 7, 3)

    def stage(t, carry):
        base = s * 320 + c * 224 + t * 32
        pltpu.sync_copy(src2.at[pl.ds(base, 32)], sidx)
        pltpu.sync_copy(dst2.at[pl.ds(base, 32)], didx)
        sdescs = [None] * 4
        gdescs = [None] * 4
        for b in range(2):
            gdescs[b] = pltpu.async_copy(y.at[sidx.at[b]], rows[b], gsem[b])
        for j in range(32):
            b = j % 4
            nj = j + 2
            if nj < 32:
                nb = nj % 4
                if sdescs[nb] is not None:
                    sdescs[nb].wait()
                gdescs[nb] = pltpu.async_copy(y.at[sidx.at[nj]], rows[nb],
                                              gsem[nb])
            gdescs[b].wait()
            sdescs[b] = pltpu.async_copy(rows[b], acc_sh.at[didx.at[j]],
                                         ssem[b], add=True)
        for b in range(4):
            if sdescs[b] is not None:
                sdescs[b].wait()
        return carry

    lax.fori_loop(0, nst, stage, 0)
    plsc.subcore_barrier()
    pltpu.sync_copy(acc_sh.at[pl.ds(s * NROWS, NROWS)],
                    sp.at[c, pl.ds(s * NROWS, NROWS)])


def _k1_body(deg_ref, x_ref, dinv_ref, y_ref):
    deg = deg_ref[0] + deg_ref[1] + 1.0
    dinv = lax.rsqrt(deg)
    dinv_ref[...] = dinv
    y_ref[...] = x_ref[...] * dinv


def _k1(deg3, x_pad):
    return pl.pallas_call(
        _k1_body,
        grid=(N_PAD // BR,),
        in_specs=[
            pl.BlockSpec((2, BR, 1), lambda i: (0, i, 0)),
            pl.BlockSpec((BR, 128), lambda i: (i, 0)),
        ],
        out_specs=[
            pl.BlockSpec((BR, 1), lambda i: (i, 0)),
            pl.BlockSpec((BR, 128), lambda i: (i, 0)),
        ],
        out_shape=[
            jax.ShapeDtypeStruct((N_PAD, 1), jnp.float32),
            jax.ShapeDtypeStruct((N_PAD, 128), jnp.float32),
        ],
    )(deg3, x_pad)


def _k2_body(sp_ref, y_ref, dinv_ref, w_ref, b_ref, out_ref):
    dinv = dinv_ref[...]
    z = dinv * (sp_ref[0] + sp_ref[1] + y_ref[...])
    h = jnp.dot(z, w_ref[...], preferred_element_type=jnp.float32) + b_ref[...]
    out_ref[...] = dinv * jnp.maximum(h, 0.0)


def _k2(sp, y, dinv2, w, b2d):
    return pl.pallas_call(
        _k2_body,
        grid=(N_PAD // BR,),
        in_specs=[
            pl.BlockSpec((2, BR, 128), lambda i: (0, i, 0)),
            pl.BlockSpec((BR, 128), lambda i: (i, 0)),
            pl.BlockSpec((BR, 1), lambda i: (i, 0)),
            pl.BlockSpec((128, 128), lambda i: (0, 0)),
            pl.BlockSpec((1, 128), lambda i: (0, 0)),
        ],
        out_specs=pl.BlockSpec((BR, 128), lambda i: (i, 0)),
        out_shape=jax.ShapeDtypeStruct((N_PAD, 128), jnp.float32),
    )(sp, y, dinv2, w, b2d)


def _k3_body(sp_ref, y_ref, dinv_ref, wm_ref, bm_ref, wl_ref, bl_ref,
             mu_ref, ls_ref):
    dinv = dinv_ref[...]
    z = dinv * (sp_ref[0] + sp_ref[1] + y_ref[...])
    mu_ref[...] = (
        jnp.dot(z, wm_ref[...], preferred_element_type=jnp.float32)
        + bm_ref[...])
    ls_ref[...] = (
        jnp.dot(z, wl_ref[...], preferred_element_type=jnp.float32)
        + bl_ref[...])


def _k3(sp, y, dinv2, wm, bm2, wl, bl2):
    return pl.pallas_call(
        _k3_body,
        grid=(N_PAD // BR,),
        in_specs=[
            pl.BlockSpec((2, BR, 128), lambda i: (0, i, 0)),
            pl.BlockSpec((BR, 128), lambda i: (i, 0)),
            pl.BlockSpec((BR, 1), lambda i: (i, 0)),
            pl.BlockSpec((128, 64), lambda i: (0, 0)),
            pl.BlockSpec((1, 64), lambda i: (0, 0)),
            pl.BlockSpec((128, 64), lambda i: (0, 0)),
            pl.BlockSpec((1, 64), lambda i: (0, 0)),
        ],
        out_specs=[
            pl.BlockSpec((BR, 64), lambda i: (i, 0)),
            pl.BlockSpec((BR, 64), lambda i: (i, 0)),
        ],
        out_shape=[
            jax.ShapeDtypeStruct((N_PAD, 64), jnp.float32),
            jax.ShapeDtypeStruct((N_PAD, 64), jnp.float32),
        ],
    )(sp, y, dinv2, wm, bm2, wl, bl2)


def kernel(x, edge_index, W1, b1, W2, b2, W_mu, b_mu, W_ls, b_ls):
    src = edge_index[0]
    dst = edge_index[1]
    fill = jnp.int32(FILL)
    src2 = jnp.full((E_PAD,), fill, jnp.int32).at[:E].set(src)
    src2 = src2.reshape(E_PAD // 64, 64)
    dst2 = jnp.full((E_PAD,), fill, jnp.int32).at[:E].set(dst)
    dst2 = dst2.reshape(E_PAD // 64, 64)
    x_pad = jnp.zeros((N_PAD, 128), jnp.float32).at[:N].set(x)
    zeros1 = jnp.zeros((NROWS,), jnp.float32)
    zrows = jnp.zeros((NROWS, 128), jnp.float32)

    degp = _deg(dst2, zeros1)
    dinv2, y0 = _k1(degp.reshape(2, N_PAD, 1), x_pad)
    s0 = _agg(src2, dst2, y0, zrows)
    y1 = _k2(s0, y0, dinv2, W1, b1.reshape(1, 128))
    s1 = _agg(src2, dst2, y1, zrows)
    y2 = _k2(s1, y1, dinv2, W2, b2.reshape(1, 128))
    s2 = _agg(src2, dst2, y2, zrows)
    mu, ls = _k3(s2, y2, dinv2, W_mu, b_mu.reshape(1, 64),
                 W_ls, b_ls.reshape(1, 64))
    return mu[:N], ls[:N]


# speedup vs baseline: 1.1354x; 1.0298x over previous
import functools

import jax
import jax.numpy as jnp
from jax import lax
from jax.experimental import pallas as pl
from jax.experimental.pallas import tpu as pltpu
from jax.experimental.pallas import tpu_sc as plsc

N = 10000
N_PAD = 10240
E = 320000
E_PAD = 327680
CPW = 80
FILL = N + 16
NROWS = N_PAD // 16
BR = 1024

_MESH = plsc.VectorSubcoreMesh(
    core_axis_name="c", subcore_axis_name="s", num_cores=2, num_subcores=16)


@functools.partial(
    pl.kernel,
    out_type=jax.ShapeDtypeStruct((2, N_PAD), jnp.float32),
    mesh=_MESH,
    scratch_types=[
        pltpu.VMEM_SHARED((N_PAD,), jnp.float32),
        pltpu.VMEM((160, 64), jnp.int32),
        pltpu.VMEM((64,), jnp.float32),
    ],
)
def _deg(dst2, zeros1, degp, deg_sh, dstbuf, ones_v):
    c = lax.axis_index("c")
    s = lax.axis_index("s")
    wid = s * 2 + c
    for j in range(4):
        ones_v[pl.ds(j * 16, 16)] = jnp.ones((16,), jnp.float32)
    pltpu.sync_copy(zeros1, deg_sh.at[pl.ds(s * NROWS, NROWS)])
    plsc.subcore_barrier()
    pltpu.sync_copy(dst2.at[pl.ds(wid * 160, 160)], dstbuf)

    def body(j, carry):
        pltpu.sync_copy(ones_v, deg_sh.at[dstbuf.at[j]], add=True)
        return carry

    lax.fori_loop(0, 160, body, 0)
    plsc.subcore_barrier()
    pltpu.sync_copy(deg_sh.at[pl.ds(s * NROWS, NROWS)],
                    degp.at[c, pl.ds(s * NROWS, NROWS)])


@functools.partial(
    pl.kernel,
    out_type=jax.ShapeDtypeStruct((2, N_PAD, 128), jnp.float32),
    mesh=_MESH,
    scratch_types=[
        pltpu.VMEM_SHARED((N_PAD, 128), jnp.float32),
        pltpu.VMEM((32, 64), jnp.int32),
        pltpu.VMEM((32, 64), jnp.int32),
        pltpu.VMEM((64, 128), jnp.float32),
        pltpu.VMEM((64, 128), jnp.float32),
        pltpu.VMEM((64, 128), jnp.float32),
        pltpu.VMEM((64, 128), jnp.float32),
        pltpu.SemaphoreType.DMA,
        pltpu.SemaphoreType.DMA,
        pltpu.SemaphoreType.DMA,
        pltpu.SemaphoreType.DMA,
        pltpu.SemaphoreType.DMA,
        pltpu.SemaphoreType.DMA,
        pltpu.SemaphoreType.DMA,
        pltpu.SemaphoreType.DMA,
    ],
)
def _agg(src2, dst2, y, zrows, sp, acc_sh, sidx, didx, r0, r1, r2, r3,
         g0, g1, g2, g3, sc0, sc1, sc2, sc3):
    c = lax.axis_index("c")
    s = lax.axis_index("s")
    rows = (r0, r1, r2, r3)
    gsem = (g0, g1, g2, g3)
    ssem = (sc0, sc1, sc2, sc3)
    pltpu.sync_copy(zrows, acc_sh.at[pl.ds(s * NROWS, NROWS)])
    plsc.subcore_barrier()

    nst = jnp.where(c == 0, 8, 2)

    def stage(t, carry):
        base = s * 320 + c * 256 + t * 32
        pltpu.sync_copy(src2.at[pl.ds(base, 32)], sidx)
        pltpu.sync_copy(dst2.at[pl.ds(base, 32)], didx)
        sdescs = [None] * 4
        gdescs = [None] * 4
        for b in range(2):
            gdescs[b] = pltpu.async_copy(y.at[sidx.at[b]], rows[b], gsem[b])
        for j in range(32):
            b = j % 4
            nj = j + 2
            if nj < 32:
                nb = nj % 4
                if sdescs[nb] is not None:
                    sdescs[nb].wait()
                gdescs[nb] = pltpu.async_copy(y.at[sidx.at[nj]], rows[nb],
                                              gsem[nb])
            gdescs[b].wait()
            sdescs[b] = pltpu.async_copy(rows[b], acc_sh.at[didx.at[j]],
                                         ssem[b], add=True)
        for b in range(4):
            if sdescs[b] is not None:
                sdescs[b].wait()
        return carry

    lax.fori_loop(0, nst, stage, 0)
    plsc.subcore_barrier()
    pltpu.sync_copy(acc_sh.at[pl.ds(s * NROWS, NROWS)],
                    sp.at[c, pl.ds(s * NROWS, NROWS)])


def _k1_body(deg_ref, x_ref, dinv_ref, y_ref):
    deg = deg_ref[0] + deg_ref[1] + 1.0
    dinv = lax.rsqrt(deg)
    dinv_ref[...] = dinv
    y_ref[...] = x_ref[...] * dinv


def _k1(deg3, x_pad):
    return pl.pallas_call(
        _k1_body,
        grid=(N_PAD // BR,),
        in_specs=[
            pl.BlockSpec((2, BR, 1), lambda i: (0, i, 0)),
            pl.BlockSpec((BR, 128), lambda i: (i, 0)),
        ],
        out_specs=[
            pl.BlockSpec((BR, 1), lambda i: (i, 0)),
            pl.BlockSpec((BR, 128), lambda i: (i, 0)),
        ],
        out_shape=[
            jax.ShapeDtypeStruct((N_PAD, 1), jnp.float32),
            jax.ShapeDtypeStruct((N_PAD, 128), jnp.float32),
        ],
    )(deg3, x_pad)


def _k2_body(sp_ref, y_ref, dinv_ref, w_ref, b_ref, out_ref):
    dinv = dinv_ref[...]
    z = dinv * (sp_ref[0] + sp_ref[1] + y_ref[...])
    h = jnp.dot(z, w_ref[...], preferred_element_type=jnp.float32) + b_ref[...]
    out_ref[...] = dinv * jnp.maximum(h, 0.0)


def _k2(sp, y, dinv2, w, b2d):
    return pl.pallas_call(
        _k2_body,
        grid=(N_PAD // BR,),
        in_specs=[
            pl.BlockSpec((2, BR, 128), lambda i: (0, i, 0)),
            pl.BlockSpec((BR, 128), lambda i: (i, 0)),
            pl.BlockSpec((BR, 1), lambda i: (i, 0)),
            pl.BlockSpec((128, 128), lambda i: (0, 0)),
            pl.BlockSpec((1, 128), lambda i: (0, 0)),
        ],
        out_specs=pl.BlockSpec((BR, 128), lambda i: (i, 0)),
        out_shape=jax.ShapeDtypeStruct((N_PAD, 128), jnp.float32),
    )(sp, y, dinv2, w, b2d)


def _k3_body(sp_ref, y_ref, dinv_ref, wm_ref, bm_ref, wl_ref, bl_ref,
             mu_ref, ls_ref):
    dinv = dinv_ref[...]
    z = dinv * (sp_ref[0] + sp_ref[1] + y_ref[...])
    mu_ref[...] = (
        jnp.dot(z, wm_ref[...], preferred_element_type=jnp.float32)
        + bm_ref[...])
    ls_ref[...] = (
        jnp.dot(z, wl_ref[...], preferred_element_type=jnp.float32)
        + bl_ref[...])


def _k3(sp, y, dinv2, wm, bm2, wl, bl2):
    return pl.pallas_call(
        _k3_body,
        grid=(N_PAD // BR,),
        in_specs=[
            pl.BlockSpec((2, BR, 128), lambda i: (0, i, 0)),
            pl.BlockSpec((BR, 128), lambda i: (i, 0)),
            pl.BlockSpec((BR, 1), lambda i: (i, 0)),
            pl.BlockSpec((128, 64), lambda i: (0, 0)),
            pl.BlockSpec((1, 64), lambda i: (0, 0)),
            pl.BlockSpec((128, 64), lambda i: (0, 0)),
            pl.BlockSpec((1, 64), lambda i: (0, 0)),
        ],
        out_specs=[
            pl.BlockSpec((BR, 64), lambda i: (i, 0)),
            pl.BlockSpec((BR, 64), lambda i: (i, 0)),
        ],
        out_shape=[
            jax.ShapeDtypeStruct((N_PAD, 64), jnp.float32),
            jax.ShapeDtypeStruct((N_PAD, 64), jnp.float32),
        ],
    )(sp, y, dinv2, wm, bm2, wl, bl2)


def kernel(x, edge_index, W1, b1, W2, b2, W_mu, b_mu, W_ls, b_ls):
    src = edge_index[0]
    dst = edge_index[1]
    fill = jnp.int32(FILL)
    src2 = jnp.full((E_PAD,), fill, jnp.int32).at[:E].set(src)
    src2 = src2.reshape(E_PAD // 64, 64)
    dst2 = jnp.full((E_PAD,), fill, jnp.int32).at[:E].set(dst)
    dst2 = dst2.reshape(E_PAD // 64, 64)
    x_pad = jnp.zeros((N_PAD, 128), jnp.float32).at[:N].set(x)
    zeros1 = jnp.zeros((NROWS,), jnp.float32)
    zrows = jnp.zeros((NROWS, 128), jnp.float32)

    degp = _deg(dst2, zeros1)
    dinv2, y0 = _k1(degp.reshape(2, N_PAD, 1), x_pad)
    s0 = _agg(src2, dst2, y0, zrows)
    y1 = _k2(s0, y0, dinv2, W1, b1.reshape(1, 128))
    s1 = _agg(src2, dst2, y1, zrows)
    y2 = _k2(s1, y1, dinv2, W2, b2.reshape(1, 128))
    s2 = _agg(src2, dst2, y2, zrows)
    mu, ls = _k3(s2, y2, dinv2, W_mu, b_mu.reshape(1, 64),
                 W_ls, b_ls.reshape(1, 64))
    return mu[:N], ls[:N]


# speedup vs baseline: 1.1786x; 1.0380x over previous
import functools

import jax
import jax.numpy as jnp
from jax import lax
from jax.experimental import pallas as pl
from jax.experimental.pallas import tpu as pltpu
from jax.experimental.pallas import tpu_sc as plsc

N = 10000
N_PAD = 10240
E = 320000
E_PAD = 327680
CPW = 80
FILL = N + 16
NROWS = N_PAD // 16
BR = 1024

_MESH = plsc.VectorSubcoreMesh(
    core_axis_name="c", subcore_axis_name="s", num_cores=2, num_subcores=16)


@functools.partial(
    pl.kernel,
    out_type=jax.ShapeDtypeStruct((2, N_PAD), jnp.float32),
    mesh=_MESH,
    scratch_types=[
        pltpu.VMEM_SHARED((N_PAD,), jnp.float32),
        pltpu.VMEM((160, 64), jnp.int32),
        pltpu.VMEM((64,), jnp.float32),
    ],
)
def _deg(dst2, zeros1, degp, deg_sh, dstbuf, ones_v):
    c = lax.axis_index("c")
    s = lax.axis_index("s")
    wid = s * 2 + c
    for j in range(4):
        ones_v[pl.ds(j * 16, 16)] = jnp.ones((16,), jnp.float32)
    pltpu.sync_copy(zeros1, deg_sh.at[pl.ds(s * NROWS, NROWS)])
    plsc.subcore_barrier()
    pltpu.sync_copy(dst2.at[pl.ds(wid * 160, 160)], dstbuf)

    def body(j, carry):
        pltpu.sync_copy(ones_v, deg_sh.at[dstbuf.at[j]], add=True)
        return carry

    lax.fori_loop(0, 160, body, 0)
    plsc.subcore_barrier()
    pltpu.sync_copy(deg_sh.at[pl.ds(s * NROWS, NROWS)],
                    degp.at[c, pl.ds(s * NROWS, NROWS)])


@functools.partial(
    pl.kernel,
    out_type=jax.ShapeDtypeStruct((2, N_PAD, 128), jnp.float32),
    mesh=_MESH,
    scratch_types=[
        pltpu.VMEM_SHARED((N_PAD, 128), jnp.float32),
        pltpu.VMEM((32, 64), jnp.int32),
        pltpu.VMEM((32, 64), jnp.int32),
        pltpu.VMEM((64, 128), jnp.float32),
        pltpu.VMEM((64, 128), jnp.float32),
        pltpu.VMEM((64, 128), jnp.float32),
        pltpu.VMEM((64, 128), jnp.float32),
        pltpu.SemaphoreType.DMA,
        pltpu.SemaphoreType.DMA,
        pltpu.SemaphoreType.DMA,
        pltpu.SemaphoreType.DMA,
        pltpu.SemaphoreType.DMA,
        pltpu.SemaphoreType.DMA,
        pltpu.SemaphoreType.DMA,
        pltpu.SemaphoreType.DMA,
    ],
)
def _agg(src2, dst2, y, zrows, sp, acc_sh, sidx, didx, r0, r1, r2, r3,
         g0, g1, g2, g3, sc0, sc1, sc2, sc3):
    c = lax.axis_index("c")
    s = lax.axis_index("s")
    rows = (r0, r1, r2, r3)
    gsem = (g0, g1, g2, g3)
    ssem = (sc0, sc1, sc2, sc3)
    pltpu.sync_copy(zrows, acc_sh.at[pl.ds(s * NROWS, NROWS)])
    plsc.subcore_barrier()

    def run_stage(base, n):
        pltpu.sync_copy(src2.at[pl.ds(base, n)], sidx.at[pl.ds(0, n)])
        pltpu.sync_copy(dst2.at[pl.ds(base, n)], didx.at[pl.ds(0, n)])
        sdescs = [None] * 4
        gdescs = [None] * 4
        for b in range(2):
            gdescs[b] = pltpu.async_copy(y.at[sidx.at[b]], rows[b], gsem[b])
        for j in range(n):
            b = j % 4
            nj = j + 2
            if nj < n:
                nb = nj % 4
                if sdescs[nb] is not None:
                    sdescs[nb].wait()
                gdescs[nb] = pltpu.async_copy(y.at[sidx.at[nj]], rows[nb],
                                              gsem[nb])
            gdescs[b].wait()
            sdescs[b] = pltpu.async_copy(rows[b], acc_sh.at[didx.at[j]],
                                         ssem[b], add=True)
        for b in range(4):
            if sdescs[b] is not None:
                sdescs[b].wait()

    nst = jnp.where(c == 0, 7, 2)

    def stage(t, carry):
        run_stage(s * 320 + c * 240 + t * 32, 32)
        return carry

    lax.fori_loop(0, nst, stage, 0)
    run_stage(s * 320 + 224 + c * 80, 16)
    plsc.subcore_barrier()
    pltpu.sync_copy(acc_sh.at[pl.ds(s * NROWS, NROWS)],
                    sp.at[c, pl.ds(s * NROWS, NROWS)])


def _k1_body(deg_ref, x_ref, dinv_ref, y_ref):
    deg = deg_ref[0] + deg_ref[1] + 1.0
    dinv = lax.rsqrt(deg)
    dinv_ref[...] = dinv
    y_ref[...] = x_ref[...] * dinv


def _k1(deg3, x_pad):
    return pl.pallas_call(
        _k1_body,
        grid=(N_PAD // BR,),
        in_specs=[
            pl.BlockSpec((2, BR, 1), lambda i: (0, i, 0)),
            pl.BlockSpec((BR, 128), lambda i: (i, 0)),
        ],
        out_specs=[
            pl.BlockSpec((BR, 1), lambda i: (i, 0)),
            pl.BlockSpec((BR, 128), lambda i: (i, 0)),
        ],
        out_shape=[
            jax.ShapeDtypeStruct((N_PAD, 1), jnp.float32),
            jax.ShapeDtypeStruct((N_PAD, 128), jnp.float32),
        ],
    )(deg3, x_pad)


def _k2_body(sp_ref, y_ref, dinv_ref, w_ref, b_ref, out_ref):
    dinv = dinv_ref[...]
    z = dinv * (sp_ref[0] + sp_ref[1] + y_ref[...])
    h = jnp.dot(z, w_ref[...], preferred_element_type=jnp.float32) + b_ref[...]
    out_ref[...] = dinv * jnp.maximum(h, 0.0)


def _k2(sp, y, dinv2, w, b2d):
    return pl.pallas_call(
        _k2_body,
        grid=(N_PAD // BR,),
        in_specs=[
            pl.BlockSpec((2, BR, 128), lambda i: (0, i, 0)),
            pl.BlockSpec((BR, 128), lambda i: (i, 0)),
            pl.BlockSpec((BR, 1), lambda i: (i, 0)),
            pl.BlockSpec((128, 128), lambda i: (0, 0)),
            pl.BlockSpec((1, 128), lambda i: (0, 0)),
        ],
        out_specs=pl.BlockSpec((BR, 128), lambda i: (i, 0)),
        out_shape=jax.ShapeDtypeStruct((N_PAD, 128), jnp.float32),
    )(sp, y, dinv2, w, b2d)


def _k3_body(sp_ref, y_ref, dinv_ref, wm_ref, bm_ref, wl_ref, bl_ref,
             mu_ref, ls_ref):
    dinv = dinv_ref[...]
    z = dinv * (sp_ref[0] + sp_ref[1] + y_ref[...])
    mu_ref[...] = (
        jnp.dot(z, wm_ref[...], preferred_element_type=jnp.float32)
        + bm_ref[...])
    ls_ref[...] = (
        jnp.dot(z, wl_ref[...], preferred_element_type=jnp.float32)
        + bl_ref[...])


def _k3(sp, y, dinv2, wm, bm2, wl, bl2):
    return pl.pallas_call(
        _k3_body,
        grid=(N_PAD // BR,),
        in_specs=[
            pl.BlockSpec((2, BR, 128), lambda i: (0, i, 0)),
            pl.BlockSpec((BR, 128), lambda i: (i, 0)),
            pl.BlockSpec((BR, 1), lambda i: (i, 0)),
            pl.BlockSpec((128, 64), lambda i: (0, 0)),
            pl.BlockSpec((1, 64), lambda i: (0, 0)),
            pl.BlockSpec((128, 64), lambda i: (0, 0)),
            pl.BlockSpec((1, 64), lambda i: (0, 0)),
        ],
        out_specs=[
            pl.BlockSpec((BR, 64), lambda i: (i, 0)),
            pl.BlockSpec((BR, 64), lambda i: (i, 0)),
        ],
        out_shape=[
            jax.ShapeDtypeStruct((N_PAD, 64), jnp.float32),
            jax.ShapeDtypeStruct((N_PAD, 64), jnp.float32),
        ],
    )(sp, y, dinv2, wm, bm2, wl, bl2)


def kernel(x, edge_index, W1, b1, W2, b2, W_mu, b_mu, W_ls, b_ls):
    src = edge_index[0]
    dst = edge_index[1]
    fill = jnp.int32(FILL)
    src2 = jnp.full((E_PAD,), fill, jnp.int32).at[:E].set(src)
    src2 = src2.reshape(E_PAD // 64, 64)
    dst2 = jnp.full((E_PAD,), fill, jnp.int32).at[:E].set(dst)
    dst2 = dst2.reshape(E_PAD // 64, 64)
    x_pad = jnp.zeros((N_PAD, 128), jnp.float32).at[:N].set(x)
    zeros1 = jnp.zeros((NROWS,), jnp.float32)
    zrows = jnp.zeros((NROWS, 128), jnp.float32)

    degp = _deg(dst2, zeros1)
    dinv2, y0 = _k1(degp.reshape(2, N_PAD, 1), x_pad)
    s0 = _agg(src2, dst2, y0, zrows)
    y1 = _k2(s0, y0, dinv2, W1, b1.reshape(1, 128))
    s1 = _agg(src2, dst2, y1, zrows)
    y2 = _k2(s1, y1, dinv2, W2, b2.reshape(1, 128))
    s2 = _agg(src2, dst2, y2, zrows)
    mu, ls = _k3(s2, y2, dinv2, W_mu, b_mu.reshape(1, 64),
                 W_ls, b_ls.reshape(1, 64))
    return mu[:N], ls[:N]


# speedup vs baseline: 1.1967x; 1.0153x over previous
import functools

import jax
import jax.numpy as jnp
from jax import lax
from jax.experimental import pallas as pl
from jax.experimental.pallas import tpu as pltpu
from jax.experimental.pallas import tpu_sc as plsc

N = 10000
N_PAD = 10240
E = 320000
E_PAD = 327680
CPW = 80
FILL = N + 16
NROWS = N_PAD // 16
BR = 1024

_MESH = plsc.VectorSubcoreMesh(
    core_axis_name="c", subcore_axis_name="s", num_cores=2, num_subcores=16)


@functools.partial(
    pl.kernel,
    out_type=jax.ShapeDtypeStruct((2, N_PAD), jnp.float32),
    mesh=_MESH,
    scratch_types=[
        pltpu.VMEM_SHARED((N_PAD,), jnp.float32),
        pltpu.VMEM((160, 64), jnp.int32),
        pltpu.VMEM((64,), jnp.float32),
    ],
)
def _deg(dst2, zeros1, degp, deg_sh, dstbuf, ones_v):
    c = lax.axis_index("c")
    s = lax.axis_index("s")
    wid = s * 2 + c
    for j in range(4):
        ones_v[pl.ds(j * 16, 16)] = jnp.ones((16,), jnp.float32)
    pltpu.sync_copy(zeros1, deg_sh.at[pl.ds(s * NROWS, NROWS)])
    plsc.subcore_barrier()
    pltpu.sync_copy(dst2.at[pl.ds(wid * 160, 160)], dstbuf)

    def body(j, carry):
        pltpu.sync_copy(ones_v, deg_sh.at[dstbuf.at[j]], add=True)
        return carry

    lax.fori_loop(0, 160, body, 0)
    plsc.subcore_barrier()
    pltpu.sync_copy(deg_sh.at[pl.ds(s * NROWS, NROWS)],
                    degp.at[c, pl.ds(s * NROWS, NROWS)])


@functools.partial(
    pl.kernel,
    out_type=jax.ShapeDtypeStruct((2, N_PAD, 128), jnp.float32),
    mesh=_MESH,
    scratch_types=[
        pltpu.VMEM_SHARED((N_PAD, 128), jnp.float32),
        pltpu.VMEM((32, 64), jnp.int32),
        pltpu.VMEM((32, 64), jnp.int32),
        pltpu.VMEM((64, 128), jnp.float32),
        pltpu.VMEM((64, 128), jnp.float32),
        pltpu.VMEM((64, 128), jnp.float32),
        pltpu.VMEM((64, 128), jnp.float32),
        pltpu.SemaphoreType.DMA,
        pltpu.SemaphoreType.DMA,
        pltpu.SemaphoreType.DMA,
        pltpu.SemaphoreType.DMA,
        pltpu.SemaphoreType.DMA,
        pltpu.SemaphoreType.DMA,
        pltpu.SemaphoreType.DMA,
        pltpu.SemaphoreType.DMA,
    ],
)
def _agg(src2, dst2, y, sp, acc_sh, sidx, didx, r0, r1, r2, r3,
         g0, g1, g2, g3, sc0, sc1, sc2, sc3):
    c = lax.axis_index("c")
    s = lax.axis_index("s")
    rows = (r0, r1, r2, r3)
    gsem = (g0, g1, g2, g3)
    ssem = (sc0, sc1, sc2, sc3)

    def zrow(r, carry):
        for j in range(8):
            r0[r, pl.ds(j * 16, 16)] = jnp.zeros((16,), jnp.float32)
        return carry

    lax.fori_loop(0, 64, zrow, 0)
    zdescs = [
        pltpu.async_copy(r0, acc_sh.at[pl.ds(s * NROWS + t * 64, 64)], g0)
        for t in range(NROWS // 64)
    ]
    for d in zdescs:
        d.wait()
    plsc.subcore_barrier()

    def run_stage(base, n):
        pltpu.sync_copy(src2.at[pl.ds(base, n)], sidx.at[pl.ds(0, n)])
        pltpu.sync_copy(dst2.at[pl.ds(base, n)], didx.at[pl.ds(0, n)])
        sdescs = [None] * 4
        gdescs = [None] * 4
        for b in range(2):
            gdescs[b] = pltpu.async_copy(y.at[sidx.at[b]], rows[b], gsem[b])
        for j in range(n):
            b = j % 4
            nj = j + 2
            if nj < n:
                nb = nj % 4
                if sdescs[nb] is not None:
                    sdescs[nb].wait()
                gdescs[nb] = pltpu.async_copy(y.at[sidx.at[nj]], rows[nb],
                                              gsem[nb])
            gdescs[b].wait()
            sdescs[b] = pltpu.async_copy(rows[b], acc_sh.at[didx.at[j]],
                                         ssem[b], add=True)
        for b in range(4):
            if sdescs[b] is not None:
                sdescs[b].wait()

    nst = jnp.where(c == 0, 7, 2)

    def stage(t, carry):
        run_stage(s * 320 + c * 240 + t * 32, 32)
        return carry

    lax.fori_loop(0, nst, stage, 0)
    run_stage(s * 320 + 224 + c * 80, 16)
    plsc.subcore_barrier()
    pltpu.sync_copy(acc_sh.at[pl.ds(s * NROWS, NROWS)],
                    sp.at[c, pl.ds(s * NROWS, NROWS)])


def _k1_body(deg_ref, x_ref, dinv_ref, y_ref):
    deg = deg_ref[0] + deg_ref[1] + 1.0
    dinv = lax.rsqrt(deg)
    dinv_ref[...] = dinv
    y_ref[...] = x_ref[...] * dinv


def _k1(deg3, x_pad):
    return pl.pallas_call(
        _k1_body,
        grid=(N_PAD // BR,),
        in_specs=[
            pl.BlockSpec((2, BR, 1), lambda i: (0, i, 0)),
            pl.BlockSpec((BR, 128), lambda i: (i, 0)),
        ],
        out_specs=[
            pl.BlockSpec((BR, 1), lambda i: (i, 0)),
            pl.BlockSpec((BR, 128), lambda i: (i, 0)),
        ],
        out_shape=[
            jax.ShapeDtypeStruct((N_PAD, 1), jnp.float32),
            jax.ShapeDtypeStruct((N_PAD, 128), jnp.float32),
        ],
    )(deg3, x_pad)


def _k2_body(sp_ref, y_ref, dinv_ref, w_ref, b_ref, out_ref):
    dinv = dinv_ref[...]
    z = dinv * (sp_ref[0] + sp_ref[1] + y_ref[...])
    h = jnp.dot(z, w_ref[...], preferred_element_type=jnp.float32) + b_ref[...]
    out_ref[...] = dinv * jnp.maximum(h, 0.0)


def _k2(sp, y, dinv2, w, b2d):
    return pl.pallas_call(
        _k2_body,
        grid=(N_PAD // BR,),
        in_specs=[
            pl.BlockSpec((2, BR, 128), lambda i: (0, i, 0)),
            pl.BlockSpec((BR, 128), lambda i: (i, 0)),
            pl.BlockSpec((BR, 1), lambda i: (i, 0)),
            pl.BlockSpec((128, 128), lambda i: (0, 0)),
            pl.BlockSpec((1, 128), lambda i: (0, 0)),
        ],
        out_specs=pl.BlockSpec((BR, 128), lambda i: (i, 0)),
        out_shape=jax.ShapeDtypeStruct((N_PAD, 128), jnp.float32),
    )(sp, y, dinv2, w, b2d)


def _k3_body(sp_ref, y_ref, dinv_ref, wm_ref, bm_ref, wl_ref, bl_ref,
             mu_ref, ls_ref):
    dinv = dinv_ref[...]
    z = dinv * (sp_ref[0] + sp_ref[1] + y_ref[...])
    mu_ref[...] = (
        jnp.dot(z, wm_ref[...], preferred_element_type=jnp.float32)
        + bm_ref[...])
    ls_ref[...] = (
        jnp.dot(z, wl_ref[...], preferred_element_type=jnp.float32)
        + bl_ref[...])


def _k3(sp, y, dinv2, wm, bm2, wl, bl2):
    return pl.pallas_call(
        _k3_body,
        grid=(N_PAD // BR,),
        in_specs=[
            pl.BlockSpec((2, BR, 128), lambda i: (0, i, 0)),
            pl.BlockSpec((BR, 128), lambda i: (i, 0)),
            pl.BlockSpec((BR, 1), lambda i: (i, 0)),
            pl.BlockSpec((128, 64), lambda i: (0, 0)),
            pl.BlockSpec((1, 64), lambda i: (0, 0)),
            pl.BlockSpec((128, 64), lambda i: (0, 0)),
            pl.BlockSpec((1, 64), lambda i: (0, 0)),
        ],
        out_specs=[
            pl.BlockSpec((BR, 64), lambda i: (i, 0)),
            pl.BlockSpec((BR, 64), lambda i: (i, 0)),
        ],
        out_shape=[
            jax.ShapeDtypeStruct((N_PAD, 64), jnp.float32),
            jax.ShapeDtypeStruct((N_PAD, 64), jnp.float32),
        ],
    )(sp, y, dinv2, wm, bm2, wl, bl2)


def kernel(x, edge_index, W1, b1, W2, b2, W_mu, b_mu, W_ls, b_ls):
    src = edge_index[0]
    dst = edge_index[1]
    fill = jnp.int32(FILL)
    src2 = jnp.full((E_PAD,), fill, jnp.int32).at[:E].set(src)
    src2 = src2.reshape(E_PAD // 64, 64)
    dst2 = jnp.full((E_PAD,), fill, jnp.int32).at[:E].set(dst)
    dst2 = dst2.reshape(E_PAD // 64, 64)
    x_pad = jnp.zeros((N_PAD, 128), jnp.float32).at[:N].set(x)
    zeros1 = jnp.zeros((NROWS,), jnp.float32)

    degp = _deg(dst2, zeros1)
    dinv2, y0 = _k1(degp.reshape(2, N_PAD, 1), x_pad)
    s0 = _agg(src2, dst2, y0)
    y1 = _k2(s0, y0, dinv2, W1, b1.reshape(1, 128))
    s1 = _agg(src2, dst2, y1)
    y2 = _k2(s1, y1, dinv2, W2, b2.reshape(1, 128))
    s2 = _agg(src2, dst2, y2)
    mu, ls = _k3(s2, y2, dinv2, W_mu, b_mu.reshape(1, 64),
                 W_ls, b_ls.reshape(1, 64))
    return mu[:N], ls[:N]
